# baseline (speedup 1.0000x reference)
import jax
import jax.numpy as jnp
from jax.experimental import pallas as pl
from jax.experimental.pallas import tpu as pltpu

IN_DIM = 2048
OUT_DIM = 2048
NL = 8
N = 16384
BLK = 512
NB = N // BLK
CD = 32 + NL * 32 + NL + 1


def _celu(x):
    return jnp.where(x > 0, x, jnp.exp(jnp.minimum(x, 0.0)) - 1.0)


def _dot(a, b):
    return jnp.dot(a, b, preferred_element_type=jnp.float32)


def _fused_kernel(idx_ref, x_ref, v_ref, t_ref, vm_ref,
                  W1_ref, b1_ref, gW2_ref, gb2_ref, W2bd_ref, b2_ref,
                  Ct_ref, Ctf_ref,
                  out_ref, G_ref, pv_ref, eacc, sacc):
    i = pl.program_id(0)

    @pl.when(i == 0)
    def _init():
        eacc[...] = jnp.zeros_like(eacc)
        sacc[...] = jnp.zeros_like(sacc)
        out_ref[...] = jnp.zeros_like(out_ref)
        Ctf = Ctf_ref[...]
        G_ref[...] = jax.lax.dot_general(
            Ctf, Ctf, dimension_numbers=(((0,), (0,)), ((), ())),
            preferred_element_type=jnp.float32)
        vm = vm_ref[...]
        rn = jax.lax.rsqrt(jnp.sum(vm * vm, axis=1, keepdims=True))
        vmn = vm * rn
        pv_ref[...] = jax.lax.dot_general(
            Ctf, vmn, dimension_numbers=(((0,), (1,)), ((), ())),
            preferred_element_type=jnp.float32)

    x = x_ref[...]
    v = v_ref[...]
    t = t_ref[...]
    idx = idx_ref[...]

    h1 = _celu(_dot(x, W1_ref[...]) + b1_ref[...])
    hg1 = h1[:, :16]
    hl1 = h1[:, 16:]

    hg2 = _celu(_dot(hg1.astype(jnp.bfloat16), gW2_ref[...])
                + gb2_ref[...])

    c16 = jax.lax.broadcasted_iota(jnp.int32, (BLK, NL * 16), 1) // 16
    hl1m = jnp.where(c16 == idx, hl1, 0.0).astype(jnp.bfloat16)
    hl2 = _celu(_dot(hl1m, W2bd_ref[...]) + b2_ref[...])
    c32 = jax.lax.broadcasted_iota(jnp.int32, (BLK, NL * 32), 1) // 32
    hl2m = jnp.where(c32 == idx, hl2, 0.0)

    oh = (jax.lax.broadcasted_iota(jnp.int32, (BLK, NL), 1)
          == idx).astype(jnp.float32)

    us = jnp.concatenate([hg2, hl2m, oh, jnp.ones((BLK, 1), jnp.float32)],
                         axis=1)
    Gb = G_ref[...]
    qg = _dot(hg2, Gb[0:32, :]) + Gb[CD - 1:CD, :]
    qs = _dot(us, Gb)
    ql = qs - qg

    ng2 = jnp.sum(qg[:, 0:32] * hg2, axis=1, keepdims=True) \
        + qg[:, CD - 1:CD]
    d = jnp.sum(qg * us, axis=1, keepdims=True) - ng2
    nl2 = jnp.sum(ql * us, axis=1, keepdims=True) - d

    w = _dot(v, Ct_ref[...])
    vs = jnp.sum(w * us, axis=1, keepdims=True)
    vv = jnp.sum(v * v, axis=1, keepdims=True)
    r = vv - 2.0 * vs + (ng2 + 2.0 * d + nl2)

    ng = jnp.sqrt(jnp.maximum(ng2, 0.0))
    nl = jnp.sqrt(jnp.maximum(nl2, 0.0))
    s_l = nl / (ng + nl)

    proj = _dot(hg2, pv_ref[0:32, :]) + pv_ref[CD - 1:CD, :]
    pm = jnp.mean(proj, axis=1, keepdims=True)
    pd = proj - pm
    stdt = jnp.sqrt(jnp.sum(pd * pd, axis=1, keepdims=True) / (NL - 1))

    cols = jnp.concatenate([d * d, r, jnp.ones_like(d)], axis=1)
    eacc[...] += jax.lax.dot_general(
        oh.astype(jnp.bfloat16), cols.astype(jnp.bfloat16),
        dimension_numbers=(((0,), (0,)), ((), ())),
        preferred_element_type=jnp.float32)

    parts = jnp.concatenate(
        [s_l, s_l * s_l, s_l * t, t, t * t, stdt],
        axis=1)
    sacc[...] += jnp.sum(parts, axis=0, keepdims=True)

    @pl.when(i == NB - 1)
    def _finalize():
        ea = eacc[...]
        sa = sacc[...]
        n = jnp.float32(N)
        cnt = ea[:, 2:3]
        loss_orth = jnp.sum(ea[:, 0:1] / cnt, keepdims=True)
        loss_recon = jnp.sum(ea[:, 1:2] / (cnt * OUT_DIM), keepdims=True)
        Sl, Sll, Slt = sa[0:1, 0:1], sa[0:1, 1:2], sa[0:1, 2:3]
        St, Stt, Sstd = sa[0:1, 3:4], sa[0:1, 4:5], sa[0:1, 5:6]
        var_t = Stt - St * St / n
        num_l = Slt - Sl * St / n
        den_l = jnp.sqrt(Sll - Sl * Sl / n) * jnp.sqrt(var_t) + 1e-8
        pcc_l = num_l / den_l
        loss_pcc = -2.0 * jnp.where(pcc_l < 0.7, pcc_l, 0.0)
        loss_bal = Sstd / n
        out_ref[...] = jnp.concatenate(
            [loss_recon, loss_orth, loss_pcc, loss_bal], axis=1)


def kernel(v, x, idx, norm_t, v_mean, gW1, gb1, gW2, gb2, gW3, gb3,
           lW1, lb1, lW2, lb2, lW3, lb3):
    W1 = jnp.concatenate([gW1, lW1.reshape(NL * 16, IN_DIM)],
                         axis=0).T.astype(jnp.bfloat16)
    b1 = jnp.concatenate([gb1, lb1.reshape(NL * 16)]).reshape(1, -1)
    gW2T = gW2.T.astype(jnp.bfloat16)
    gb2r = gb2.reshape(1, -1)
    W2bd = jax.scipy.linalg.block_diag(
        *[lW2[e].T for e in range(NL)]).astype(jnp.bfloat16)
    b2 = lb2.reshape(1, NL * 32)
    lW3t = jnp.transpose(lW3, (1, 0, 2)).reshape(OUT_DIM, NL * 32)
    Ct_f = jnp.concatenate(
        [gW3, lW3t, lb3.T, gb3.reshape(OUT_DIM, 1)],
        axis=1)
    Ct_bf = Ct_f.astype(jnp.bfloat16)
    idx2 = idx.reshape(N, 1).astype(jnp.int32)

    row = lambda i: (i, 0)
    rep = lambda i: (0, 0)
    out = pl.pallas_call(
        _fused_kernel,
        grid=(NB,),
        in_specs=[
            pl.BlockSpec((BLK, 1), row),
            pl.BlockSpec((BLK, IN_DIM), row),
            pl.BlockSpec((BLK, OUT_DIM), row),
            pl.BlockSpec((BLK, 1), row),
            pl.BlockSpec((NL, OUT_DIM), rep),
            pl.BlockSpec((IN_DIM, 16 + NL * 16), rep),
            pl.BlockSpec((1, 16 + NL * 16), rep),
            pl.BlockSpec((16, 32), rep),
            pl.BlockSpec((1, 32), rep),
            pl.BlockSpec((NL * 16, NL * 32), rep),
            pl.BlockSpec((1, NL * 32), rep),
            pl.BlockSpec((OUT_DIM, CD), rep),
            pl.BlockSpec((OUT_DIM, CD), rep),
        ],
        out_specs=pl.BlockSpec((1, 4), rep),
        out_shape=jax.ShapeDtypeStruct((1, 4), jnp.float32),
        scratch_shapes=[
            pltpu.VMEM((CD, CD), jnp.float32),
            pltpu.VMEM((CD, NL), jnp.float32),
            pltpu.VMEM((NL, 3), jnp.float32),
            pltpu.VMEM((1, 6), jnp.float32),
        ],
        compiler_params=pltpu.CompilerParams(
            dimension_semantics=("arbitrary",),
        ),
    )(idx2, x, v, norm_t, v_mean,
      W1, b1, gW2T, gb2r, W2bd, b2, Ct_bf, Ct_f)
    return (out[0, 0], out[0, 1], out[0, 2], out[0, 3])


# speedup vs baseline: 1.0063x; 1.0063x over previous
import jax
import jax.numpy as jnp
from jax.experimental import pallas as pl
from jax.experimental.pallas import tpu as pltpu

IN_DIM = 2048
OUT_DIM = 2048
NL = 8
N = 16384
BLK = 512
NB = N // BLK
CD = 32 + NL * 32 + NL + 1


def _celu(x):
    return jnp.where(x > 0, x, jnp.exp(jnp.minimum(x, 0.0)) - 1.0)


def _dot(a, b):
    return jnp.dot(a, b, preferred_element_type=jnp.float32)


def _fused_kernel(idx_ref, x_ref, v_ref, t_ref, vm_ref,
                  W1_ref, b1_ref, gW2_ref, gb2_ref, W2bd_ref, b2_ref,
                  Ct_ref, Ctf_ref,
                  out_ref, G_ref, pv_ref, eacc, sacc):
    i = pl.program_id(0)

    @pl.when(i == 0)
    def _init():
        eacc[...] = jnp.zeros_like(eacc)
        sacc[...] = jnp.zeros_like(sacc)
        out_ref[...] = jnp.zeros_like(out_ref)
        Ctf = Ctf_ref[...]
        G_ref[...] = jax.lax.dot_general(
            Ctf, Ctf, dimension_numbers=(((0,), (0,)), ((), ())),
            preferred_element_type=jnp.float32)
        vm = vm_ref[...]
        rn = jax.lax.rsqrt(jnp.sum(vm * vm, axis=1, keepdims=True))
        vmn = vm * rn
        pv_ref[...] = jax.lax.dot_general(
            Ctf, vmn, dimension_numbers=(((0,), (1,)), ((), ())),
            preferred_element_type=jnp.float32)

    x = x_ref[...]
    v = v_ref[...]
    t = t_ref[...]
    idx = idx_ref[...]

    h1 = _celu(_dot(x, W1_ref[...]) + b1_ref[...])
    hg1 = h1[:, :16]
    hl1 = h1[:, 16:]

    hg2 = _celu(_dot(hg1, gW2_ref[...]) + gb2_ref[...])

    c16 = jax.lax.broadcasted_iota(jnp.int32, (BLK, NL * 16), 1) // 16
    hl1m = jnp.where(c16 == idx, hl1, 0.0)
    hl2 = _celu(_dot(hl1m, W2bd_ref[...]) + b2_ref[...])
    c32 = jax.lax.broadcasted_iota(jnp.int32, (BLK, NL * 32), 1) // 32
    hl2m = jnp.where(c32 == idx, hl2, 0.0)

    oh = (jax.lax.broadcasted_iota(jnp.int32, (BLK, NL), 1)
          == idx).astype(jnp.float32)

    us = jnp.concatenate([hg2, hl2m, oh, jnp.ones((BLK, 1), jnp.float32)],
                         axis=1)
    Gb = G_ref[...]
    qg = _dot(hg2, Gb[0:32, :]) + Gb[CD - 1:CD, :]
    qs = _dot(us, Gb)
    ql = qs - qg

    ng2 = jnp.sum(qg[:, 0:32] * hg2, axis=1, keepdims=True) \
        + qg[:, CD - 1:CD]
    d = jnp.sum(qg * us, axis=1, keepdims=True) - ng2
    nl2 = jnp.sum(ql * us, axis=1, keepdims=True) - d

    w = _dot(v, Ct_ref[...])
    vs = jnp.sum(w * us, axis=1, keepdims=True)
    vv = jnp.sum(v * v, axis=1, keepdims=True)
    r = vv - 2.0 * vs + (ng2 + 2.0 * d + nl2)

    ng = jnp.sqrt(jnp.maximum(ng2, 0.0))
    nl = jnp.sqrt(jnp.maximum(nl2, 0.0))
    s_l = nl / (ng + nl)

    proj = _dot(hg2, pv_ref[0:32, :]) + pv_ref[CD - 1:CD, :]
    pm = jnp.mean(proj, axis=1, keepdims=True)
    pd = proj - pm
    stdt = jnp.sqrt(jnp.sum(pd * pd, axis=1, keepdims=True) / (NL - 1))

    cols = jnp.concatenate([d * d, r, jnp.ones_like(d)], axis=1)
    eacc[...] += jax.lax.dot_general(
        oh.astype(jnp.bfloat16), cols.astype(jnp.bfloat16),
        dimension_numbers=(((0,), (0,)), ((), ())),
        preferred_element_type=jnp.float32)

    parts = jnp.concatenate(
        [s_l, s_l * s_l, s_l * t, t, t * t, stdt],
        axis=1)
    sacc[...] += jnp.sum(parts, axis=0, keepdims=True)

    @pl.when(i == NB - 1)
    def _finalize():
        ea = eacc[...]
        sa = sacc[...]
        n = jnp.float32(N)
        cnt = ea[:, 2:3]
        loss_orth = jnp.sum(ea[:, 0:1] / cnt, keepdims=True)
        loss_recon = jnp.sum(ea[:, 1:2] / (cnt * OUT_DIM), keepdims=True)
        Sl, Sll, Slt = sa[0:1, 0:1], sa[0:1, 1:2], sa[0:1, 2:3]
        St, Stt, Sstd = sa[0:1, 3:4], sa[0:1, 4:5], sa[0:1, 5:6]
        var_t = Stt - St * St / n
        num_l = Slt - Sl * St / n
        den_l = jnp.sqrt(Sll - Sl * Sl / n) * jnp.sqrt(var_t) + 1e-8
        pcc_l = num_l / den_l
        loss_pcc = -2.0 * jnp.where(pcc_l < 0.7, pcc_l, 0.0)
        loss_bal = Sstd / n
        out_ref[...] = jnp.concatenate(
            [loss_recon, loss_orth, loss_pcc, loss_bal], axis=1)


def kernel(v, x, idx, norm_t, v_mean, gW1, gb1, gW2, gb2, gW3, gb3,
           lW1, lb1, lW2, lb2, lW3, lb3):
    W1 = jnp.concatenate([gW1, lW1.reshape(NL * 16, IN_DIM)], axis=0).T
    b1 = jnp.concatenate([gb1, lb1.reshape(NL * 16)]).reshape(1, -1)
    gW2T = gW2.T
    gb2r = gb2.reshape(1, -1)
    W2bd = jax.scipy.linalg.block_diag(*[lW2[e].T for e in range(NL)])
    b2 = lb2.reshape(1, NL * 32)
    lW3t = jnp.transpose(lW3, (1, 0, 2)).reshape(OUT_DIM, NL * 32)
    Ct_f = jnp.concatenate(
        [gW3, lW3t, lb3.T, gb3.reshape(OUT_DIM, 1)],
        axis=1)
    Ct_bf = Ct_f.astype(jnp.bfloat16)
    idx2 = idx.reshape(N, 1).astype(jnp.int32)

    row = lambda i: (i, 0)
    rep = lambda i: (0, 0)
    out = pl.pallas_call(
        _fused_kernel,
        grid=(NB,),
        in_specs=[
            pl.BlockSpec((BLK, 1), row),
            pl.BlockSpec((BLK, IN_DIM), row),
            pl.BlockSpec((BLK, OUT_DIM), row),
            pl.BlockSpec((BLK, 1), row),
            pl.BlockSpec((NL, OUT_DIM), rep),
            pl.BlockSpec((IN_DIM, 16 + NL * 16), rep),
            pl.BlockSpec((1, 16 + NL * 16), rep),
            pl.BlockSpec((16, 32), rep),
            pl.BlockSpec((1, 32), rep),
            pl.BlockSpec((NL * 16, NL * 32), rep),
            pl.BlockSpec((1, NL * 32), rep),
            pl.BlockSpec((OUT_DIM, CD), rep),
            pl.BlockSpec((OUT_DIM, CD), rep),
        ],
        out_specs=pl.BlockSpec((1, 4), rep),
        out_shape=jax.ShapeDtypeStruct((1, 4), jnp.float32),
        scratch_shapes=[
            pltpu.VMEM((CD, CD), jnp.float32),
            pltpu.VMEM((CD, NL), jnp.float32),
            pltpu.VMEM((NL, 3), jnp.float32),
            pltpu.VMEM((1, 6), jnp.float32),
        ],
        compiler_params=pltpu.CompilerParams(
            dimension_semantics=("arbitrary",),
        ),
    )(idx2, x, v, norm_t, v_mean,
      W1, b1, gW2T, gb2r, W2bd, b2, Ct_bf, Ct_f)
    return (out[0, 0], out[0, 1], out[0, 2], out[0, 3])
